# initial kernel scaffold (unmeasured)
import functools

import jax
import jax.numpy as jnp
from jax import lax
from jax.experimental import pallas as pl
from jax.experimental.pallas import tpu as pltpu

N_DEV = 4


def kernel(x, w_mat):
    m_global, k_per = x.shape
    k_per2, n = w_mat.shape
    assert k_per == k_per2
    m_per = m_global // N_DEV

    def body(x_ref, w_ref, out_ref, comm_ref, send_sems, recv_sems, credit_sem):
        my = lax.axis_index("i")
        left = (my - 1 + N_DEV) % N_DEV
        right = (my + 1) % N_DEV

        barrier_sem = pltpu.get_barrier_semaphore()
        for nbr in (left, right):
            pl.semaphore_signal(
                barrier_sem, inc=1,
                device_id=(nbr,), device_id_type=pl.DeviceIdType.MESH,
            )
        pl.semaphore_wait(barrier_sem, 2)

        def local_chunk(c):
            return jnp.dot(
                x_ref[pl.ds(c * m_per, m_per), :], w_ref[:, :],
                preferred_element_type=jnp.float32,
            )

        comm_ref[0] = local_chunk((my - 1 + N_DEV) % N_DEV)

        for h in range(N_DEV - 1):
            send_slot = h % 2
            recv_slot = (h + 1) % 2
            if h >= 1:
                pl.semaphore_wait(credit_sem, 1)
            rdma = pltpu.make_async_remote_copy(
                src_ref=comm_ref.at[send_slot],
                dst_ref=comm_ref.at[recv_slot],
                send_sem=send_sems.at[h],
                recv_sem=recv_sems.at[h],
                device_id=(right,),
                device_id_type=pl.DeviceIdType.MESH,
            )
            rdma.start()
            local = local_chunk((my - 2 - h + 2 * N_DEV) % N_DEV)
            rdma.wait()
            if h < N_DEV - 2:
                pl.semaphore_signal(
                    credit_sem, inc=1,
                    device_id=(left,), device_id_type=pl.DeviceIdType.MESH,
                )
                comm_ref[recv_slot] = comm_ref[recv_slot] + local
            else:
                out_ref[:, :] = comm_ref[recv_slot] + local

    return pl.pallas_call(
        body,
        out_shape=jax.ShapeDtypeStruct((m_per, n), jnp.float32),
        in_specs=[
            pl.BlockSpec(memory_space=pltpu.VMEM),
            pl.BlockSpec(memory_space=pltpu.VMEM),
        ],
        out_specs=pl.BlockSpec(memory_space=pltpu.VMEM),
        scratch_shapes=[
            pltpu.VMEM((2, m_per, n), jnp.float32),
            pltpu.SemaphoreType.DMA((N_DEV - 1,)),
            pltpu.SemaphoreType.DMA((N_DEV - 1,)),
            pltpu.SemaphoreType.REGULAR,
        ],
        compiler_params=pltpu.CompilerParams(collective_id=0),
    )(x, w_mat)


# baseline (device time: 303511 ns/iter reference)
import functools

import jax
import jax.numpy as jnp
from jax import lax
from jax.experimental import pallas as pl
from jax.experimental.pallas import tpu as pltpu

N_DEV = 4


def kernel(x, w_mat):
    m_global, k_per = x.shape
    k_per2, n = w_mat.shape
    assert k_per == k_per2
    m_per = m_global // N_DEV

    def body(x_ref, w_ref, out_ref, comm_ref, send_sems, recv_sems, credit_sem):
        my = lax.axis_index("i")
        left = (my - 1 + N_DEV) % N_DEV
        right = (my + 1) % N_DEV

        barrier_sem = pltpu.get_barrier_semaphore()
        for nbr in (left, right):
            pl.semaphore_signal(
                barrier_sem, inc=1,
                device_id=(nbr,), device_id_type=pl.DeviceIdType.MESH,
            )
        pl.semaphore_wait(barrier_sem, 2)

        def local_chunk(c):
            return jnp.dot(
                x_ref[pl.ds(c * m_per, m_per), :], w_ref[:, :],
                preferred_element_type=jnp.float32,
            )

        comm_ref[0] = local_chunk((my - 1 + N_DEV) % N_DEV)

        for h in range(N_DEV - 1):
            send_slot = h % 2
            recv_slot = (h + 1) % 2
            if h >= 1:
                pl.semaphore_wait(credit_sem, 1)
            rdma = pltpu.make_async_remote_copy(
                src_ref=comm_ref.at[send_slot],
                dst_ref=comm_ref.at[recv_slot],
                send_sem=send_sems.at[h],
                recv_sem=recv_sems.at[h],
                device_id=(right,),
                device_id_type=pl.DeviceIdType.MESH,
            )
            rdma.start()
            local = local_chunk((my - 2 - h + 2 * N_DEV) % N_DEV)
            rdma.wait()
            if h < N_DEV - 2:
                pl.semaphore_signal(
                    credit_sem, inc=1,
                    device_id=(left,), device_id_type=pl.DeviceIdType.MESH,
                )
                comm_ref[recv_slot] = comm_ref[recv_slot] + local
            else:
                out_ref[:, :] = comm_ref[recv_slot] + local

    return pl.pallas_call(
        body,
        out_shape=jax.ShapeDtypeStruct((m_per, n), jnp.float32),
        in_specs=[
            pl.BlockSpec(memory_space=pltpu.VMEM),
            pl.BlockSpec(memory_space=pltpu.VMEM),
        ],
        out_specs=pl.BlockSpec(memory_space=pltpu.VMEM),
        scratch_shapes=[
            pltpu.VMEM((2, m_per, n), jnp.float32),
            pltpu.SemaphoreType.DMA((N_DEV - 1,)),
            pltpu.SemaphoreType.DMA((N_DEV - 1,)),
            pltpu.SemaphoreType.REGULAR,
        ],
        compiler_params=pltpu.CompilerParams(
            collective_id=0,
            vmem_limit_bytes=100 * 1024 * 1024,
        ),
    )(x, w_mat)


# device time: 168607 ns/iter; 1.8001x vs baseline; 1.8001x over previous
import jax
import jax.numpy as jnp
from jax import lax
from jax.experimental import pallas as pl
from jax.experimental.pallas import tpu as pltpu

N_DEV = 4


def kernel(x, w_mat):
    m_global, k_per = x.shape
    k_per2, n = w_mat.shape
    assert k_per == k_per2
    m_per = m_global // N_DEV
    n_half = n // 2

    def body(x_ref, w_ref, out_ref, commR, commL,
             sendR_sems, recvR_sems, sendL_sems, recvL_sems,
             creditR, creditL):
        my = lax.axis_index("i")
        left = (my - 1 + N_DEV) % N_DEV
        right = (my + 1) % N_DEV

        barrier_sem = pltpu.get_barrier_semaphore()
        for nbr in (left, right):
            pl.semaphore_signal(
                barrier_sem, inc=1,
                device_id=(nbr,), device_id_type=pl.DeviceIdType.MESH,
            )
        pl.semaphore_wait(barrier_sem, 2)

        def chunk_lo(c):
            return jnp.dot(
                x_ref[pl.ds(c * m_per, m_per), :], w_ref[:, :n_half],
                preferred_element_type=jnp.float32,
            )

        def chunk_hi(c):
            return jnp.dot(
                x_ref[pl.ds(c * m_per, m_per), :], w_ref[:, n_half:],
                preferred_element_type=jnp.float32,
            )

        commR[0] = chunk_lo((my - 1 + N_DEV) % N_DEV)
        commL[0] = chunk_hi((my + 1) % N_DEV)

        for h in range(N_DEV - 1):
            send_slot = h % 2
            recv_slot = (h + 1) % 2
            if h >= 1:
                pl.semaphore_wait(creditR, 1)
                pl.semaphore_wait(creditL, 1)
            rdmaR = pltpu.make_async_remote_copy(
                src_ref=commR.at[send_slot],
                dst_ref=commR.at[recv_slot],
                send_sem=sendR_sems.at[h],
                recv_sem=recvR_sems.at[h],
                device_id=(right,),
                device_id_type=pl.DeviceIdType.MESH,
            )
            rdmaL = pltpu.make_async_remote_copy(
                src_ref=commL.at[send_slot],
                dst_ref=commL.at[recv_slot],
                send_sem=sendL_sems.at[h],
                recv_sem=recvL_sems.at[h],
                device_id=(left,),
                device_id_type=pl.DeviceIdType.MESH,
            )
            rdmaR.start()
            rdmaL.start()
            loc_r = chunk_lo((my - 2 - h + 2 * N_DEV) % N_DEV)
            loc_l = chunk_hi((my + 2 + h) % N_DEV)
            rdmaR.wait()
            rdmaL.wait()
            if h < N_DEV - 2:
                pl.semaphore_signal(
                    creditR, inc=1,
                    device_id=(left,), device_id_type=pl.DeviceIdType.MESH,
                )
                pl.semaphore_signal(
                    creditL, inc=1,
                    device_id=(right,), device_id_type=pl.DeviceIdType.MESH,
                )
                commR[recv_slot] = commR[recv_slot] + loc_r
                commL[recv_slot] = commL[recv_slot] + loc_l
            else:
                out_ref[:, :n_half] = commR[recv_slot] + loc_r
                out_ref[:, n_half:] = commL[recv_slot] + loc_l

    return pl.pallas_call(
        body,
        out_shape=jax.ShapeDtypeStruct((m_per, n), jnp.float32),
        in_specs=[
            pl.BlockSpec(memory_space=pltpu.VMEM),
            pl.BlockSpec(memory_space=pltpu.VMEM),
        ],
        out_specs=pl.BlockSpec(memory_space=pltpu.VMEM),
        scratch_shapes=[
            pltpu.VMEM((2, m_per, n_half), jnp.float32),
            pltpu.VMEM((2, m_per, n_half), jnp.float32),
            pltpu.SemaphoreType.DMA((N_DEV - 1,)),
            pltpu.SemaphoreType.DMA((N_DEV - 1,)),
            pltpu.SemaphoreType.DMA((N_DEV - 1,)),
            pltpu.SemaphoreType.DMA((N_DEV - 1,)),
            pltpu.SemaphoreType.REGULAR,
            pltpu.SemaphoreType.REGULAR,
        ],
        compiler_params=pltpu.CompilerParams(
            collective_id=0,
            vmem_limit_bytes=100 * 1024 * 1024,
        ),
    )(x, w_mat)


# device time: 161009 ns/iter; 1.8851x vs baseline; 1.0472x over previous
import jax
import jax.numpy as jnp
from jax import lax
from jax.experimental import pallas as pl
from jax.experimental.pallas import tpu as pltpu

N_DEV = 4
N_HOP = N_DEV - 1
T = 2


def kernel(x, w_mat):
    m_global, k_per = x.shape
    k_per2, n = w_mat.shape
    assert k_per == k_per2
    m_per = m_global // N_DEV
    n_half = n // 2
    tile = n_half // T

    def body(x_ref, w_ref, out_ref, commR, commL,
             sR_sems, rR_sems, sL_sems, rL_sems):
        my = lax.axis_index("i")
        left = (my - 1 + N_DEV) % N_DEV
        right = (my + 1) % N_DEV

        barrier_sem = pltpu.get_barrier_semaphore()
        for nbr in (left, right):
            pl.semaphore_signal(
                barrier_sem, inc=1,
                device_id=(nbr,), device_id_type=pl.DeviceIdType.MESH,
            )
        pl.semaphore_wait(barrier_sem, 2)

        def loc(c, col0):
            return jnp.dot(
                x_ref[pl.ds(c * m_per, m_per), :],
                w_ref[:, col0:col0 + tile],
                preferred_element_type=jnp.float32,
            )

        def mk(ring, h, t):
            comm, ssem, rsem, dst, base = {
                "R": (commR, sR_sems, rR_sems, right, 0),
                "L": (commL, sL_sems, rL_sems, left, n_half),
            }[ring]
            if h < N_HOP - 1:
                dst_ref = comm.at[h + 1, t]
            else:
                col0 = base + t * tile
                dst_ref = out_ref.at[:, pl.ds(col0, tile)]
            return pltpu.make_async_remote_copy(
                src_ref=comm.at[h, t],
                dst_ref=dst_ref,
                send_sem=ssem.at[h, t],
                recv_sem=rsem.at[h, t],
                device_id=(dst,),
                device_id_type=pl.DeviceIdType.MESH,
            )

        descs = {}

        cR0 = (my - 1 + N_DEV) % N_DEV
        cL0 = (my + 1) % N_DEV
        for t in range(T):
            commR[0, t] = loc(cR0, t * tile)
            d = descs[("R", 0, t)] = mk("R", 0, t)
            d.start()
            commL[0, t] = loc(cL0, n_half + t * tile)
            d = descs[("L", 0, t)] = mk("L", 0, t)
            d.start()

        for h in range(N_HOP):
            cR = (my - 2 - h + 2 * N_DEV) % N_DEV
            cL = (my + 2 + h) % N_DEV
            for t in range(T):
                for ring, c, col0 in (
                    ("R", cR, t * tile),
                    ("L", cL, n_half + t * tile),
                ):
                    comm = commR if ring == "R" else commL
                    l = loc(c, col0)
                    descs[(ring, h, t)].wait_recv()
                    if h < N_HOP - 1:
                        comm[h + 1, t] = comm[h + 1, t] + l
                        d = descs[(ring, h + 1, t)] = mk(ring, h + 1, t)
                        d.start()
                    else:
                        out_ref[:, col0:col0 + tile] = (
                            out_ref[:, col0:col0 + tile] + l
                        )

        for d in descs.values():
            d.wait_send()

    return pl.pallas_call(
        body,
        out_shape=jax.ShapeDtypeStruct((m_per, n), jnp.float32),
        in_specs=[
            pl.BlockSpec(memory_space=pltpu.VMEM),
            pl.BlockSpec(memory_space=pltpu.VMEM),
        ],
        out_specs=pl.BlockSpec(memory_space=pltpu.VMEM),
        scratch_shapes=[
            pltpu.VMEM((N_HOP, T, m_per, tile), jnp.float32),
            pltpu.VMEM((N_HOP, T, m_per, tile), jnp.float32),
            pltpu.SemaphoreType.DMA((N_HOP, T)),
            pltpu.SemaphoreType.DMA((N_HOP, T)),
            pltpu.SemaphoreType.DMA((N_HOP, T)),
            pltpu.SemaphoreType.DMA((N_HOP, T)),
        ],
        compiler_params=pltpu.CompilerParams(
            collective_id=0,
            vmem_limit_bytes=100 * 1024 * 1024,
        ),
    )(x, w_mat)
